# baseline (speedup 1.0000x reference)
import jax
import jax.numpy as jnp
from jax.experimental import pallas as pl

N = 10000
H = 8
C = 32


def _ln(x, g, b, eps=1e-5):
    m = x.mean(-1, keepdims=True)
    v = ((x - m) ** 2).mean(-1, keepdims=True)
    return (x - m) / jnp.sqrt(v + eps) * g + b


def _gat(x, src, dst, p):
    xl = (x @ p["gat_w"]).reshape(N, H, C)
    a_s = (xl * p["att_src"]).sum(-1)
    a_d = (xl * p["att_dst"]).sum(-1)
    loop = jnp.arange(N, dtype=src.dtype)
    s = jnp.concatenate([src, loop])
    d = jnp.concatenate([dst, loop])
    alpha = jax.nn.leaky_relu(a_s[s] + a_d[d], negative_slope=0.2)
    amax = jax.lax.stop_gradient(jax.ops.segment_max(alpha, d, num_segments=N))
    ex = jnp.exp(alpha - amax[d])
    den = jax.ops.segment_sum(ex, d, num_segments=N)
    w = ex / (den[d] + 1e-16)
    msg = xl[s] * w[:, :, None]
    out = jax.ops.segment_sum(msg, d, num_segments=N)
    return out.reshape(N, H * C) + p["gat_b"]


def _enc(x, src, dst, p):
    h = _ln(x + _gat(x, src, dst, p), p["n1_g"], p["n1_b"])
    hf = jax.nn.relu(h @ p["ff_w1"] + p["ff_b1"]) @ p["ff_w2"] + p["ff_b2"]
    return _ln(h + hf, p["n2_g"], p["n2_b"])


def _pred_mm_kernel(ef_ref, w1_ref, b1_ref, w2_ref, b2_ref, o_ref):
    z = jnp.maximum(ef_ref[...] @ w1_ref[...] + b1_ref[...], 0.0)
    o_ref[...] = jnp.maximum(z @ w2_ref[...] + b2_ref[...], 0.0)


def kernel(x, edge_index, virtual_edge_index, params):
    h = _ln(jax.nn.relu(x @ params["pre_w"] + params["pre_b"]),
            params["pre_g"], params["pre_bt"])
    for p in params["v"]:
        h = _enc(h, virtual_edge_index[0], virtual_edge_index[1], p)
    for p in params["r"]:
        h = _enc(h, edge_index[0], edge_index[1], p)
    ef = jnp.concatenate([h[edge_index[0]], h[edge_index[1]]], axis=-1)
    pp = params["pred"]
    E = ef.shape[0]
    BLK = 4000
    flow = pl.pallas_call(
        _pred_mm_kernel,
        grid=(E // BLK,),
        in_specs=[
            pl.BlockSpec((BLK, 512), lambda i: (i, 0)),
            pl.BlockSpec((512, 256), lambda i: (0, 0)),
            pl.BlockSpec((256,), lambda i: (0,)),
            pl.BlockSpec((256, 1), lambda i: (0, 0)),
            pl.BlockSpec((1,), lambda i: (0,)),
        ],
        out_specs=pl.BlockSpec((BLK, 1), lambda i: (i, 0)),
        out_shape=jax.ShapeDtypeStruct((E, 1), jnp.float32),
    )(ef, pp["w1"], pp["b1"], pp["w2"], pp["b2"])
    return flow[:, 0]


# speedup vs baseline: 20.9734x; 20.9734x over previous
import functools

import jax
import jax.numpy as jnp
from jax import lax
from jax.experimental import pallas as pl
from jax.experimental.pallas import tpu as pltpu
from jax.experimental.pallas import tpu_sc as plsc

N = 10000
D_IN = 128
D = 256
H = 8
C = 32
FF = 512
PRED = 256
E_R = 320000
E_V = 160000

NC = 2
NS = 16
L = 16

BLK = 1000


def _ln(z, g, b, eps=1e-5):
    m = jnp.mean(z, axis=-1, keepdims=True)
    v = jnp.mean((z - m) ** 2, axis=-1, keepdims=True)
    return (z - m) * jax.lax.rsqrt(v + eps) * g + b


def _pre_body(x_ref, w_ref, b_ref, g_ref, bt_ref, o_ref):
    z = jnp.maximum(x_ref[...] @ w_ref[...] + b_ref[...], 0.0)
    o_ref[...] = _ln(z, g_ref[...], bt_ref[...])


def _pre_call(x, pw, pb, pg, pbt):
    return pl.pallas_call(
        _pre_body,
        grid=(N // BLK,),
        in_specs=[
            pl.BlockSpec((BLK, D_IN), lambda i: (i, 0)),
            pl.BlockSpec((D_IN, D), lambda i: (0, 0)),
            pl.BlockSpec((D,), lambda i: (0,)),
            pl.BlockSpec((D,), lambda i: (0,)),
            pl.BlockSpec((D,), lambda i: (0,)),
        ],
        out_specs=pl.BlockSpec((BLK, D), lambda i: (i, 0)),
        out_shape=jax.ShapeDtypeStruct((N, D), jnp.float32),
    )(x, pw, pb, pg, pbt)


def _gat_pre_body(h_ref, w_ref, asrc_ref, adst_ref,
                  xl2_ref, asd_ref, asdsw_ref, den0_ref, sm_ref):
    xl = h_ref[...] @ w_ref[...]
    asrc = asrc_ref[...]
    adst = adst_ref[...]
    a_s_parts = []
    a_d_parts = []
    sm_parts = []
    for hh in range(H):
        xs = xl[:, hh * C:(hh + 1) * C]
        a_s_parts.append(jnp.sum(xs * asrc[hh * C:(hh + 1) * C][None, :],
                                 axis=1, keepdims=True))
        a_d_parts.append(jnp.sum(xs * adst[hh * C:(hh + 1) * C][None, :],
                                 axis=1, keepdims=True))
    a_s = jnp.concatenate(a_s_parts, axis=1)
    a_d = jnp.concatenate(a_d_parts, axis=1)
    al = a_s + a_d
    exl = jnp.exp(jnp.maximum(al, 0.2 * al))
    for hh in range(H):
        sm_parts.append(exl[:, hh:hh + 1] * xl[:, hh * C:(hh + 1) * C])
    sm = jnp.concatenate(sm_parts, axis=1)
    zeros8 = jnp.zeros_like(exl)
    exl16 = jnp.concatenate([exl, zeros8], axis=1)
    asd_ref[...] = jnp.concatenate([a_s, a_d], axis=1)
    asdsw_ref[...] = jnp.concatenate([a_d, a_s], axis=1)
    xl2_ref[...] = jnp.concatenate([xl[None, :, :128], xl[None, :, 128:]], 0)
    den0_ref[...] = jnp.concatenate([exl16[None], jnp.zeros_like(exl16)[None]], 0)
    sm_ref[...] = jnp.concatenate([sm[None, :, :128], sm[None, :, 128:]], 0)


def _gat_pre_call(h, gw, asrc, adst):
    return pl.pallas_call(
        _gat_pre_body,
        grid=(N // BLK,),
        in_specs=[
            pl.BlockSpec((BLK, D), lambda i: (i, 0)),
            pl.BlockSpec((D, D), lambda i: (0, 0)),
            pl.BlockSpec((D,), lambda i: (0,)),
            pl.BlockSpec((D,), lambda i: (0,)),
        ],
        out_specs=[
            pl.BlockSpec((2, BLK, 128), lambda i: (0, i, 0)),
            pl.BlockSpec((BLK, 16), lambda i: (i, 0)),
            pl.BlockSpec((BLK, 16), lambda i: (i, 0)),
            pl.BlockSpec((2, BLK, 16), lambda i: (0, i, 0)),
            pl.BlockSpec((2, BLK, 128), lambda i: (0, i, 0)),
        ],
        out_shape=[
            jax.ShapeDtypeStruct((2, N, 128), jnp.float32),
            jax.ShapeDtypeStruct((N, 16), jnp.float32),
            jax.ShapeDtypeStruct((N, 16), jnp.float32),
            jax.ShapeDtypeStruct((2, N, 16), jnp.float32),
            jax.ShapeDtypeStruct((2, N, 128), jnp.float32),
        ],
    )(h, gw, asrc, adst)


def _sc_att_call(asd, asdsw, deninit, src, dst, E, CH):
    PW = E // (NC * NS)
    chunks = PW // CH
    mesh = plsc.VectorSubcoreMesh(core_axis_name="c", subcore_axis_name="s")

    @functools.partial(
        pl.kernel,
        out_type=[
            jax.ShapeDtypeStruct((E, 16), jnp.float32),
            jax.ShapeDtypeStruct((NC, N, 16), jnp.float32),
        ],
        mesh=mesh,
        compiler_params=pltpu.CompilerParams(use_tc_tiling_on_sc=False, needs_layout_passes=False),
        scratch_types=[
            pltpu.VMEM((1, CH), jnp.int32),
            pltpu.VMEM((1, CH), jnp.int32),
            pltpu.VMEM((CH, 16), jnp.float32),
            pltpu.VMEM((CH, 16), jnp.float32),
            pltpu.VMEM((CH, 16), jnp.float32),
            pltpu.VMEM_SHARED((N, 16), jnp.float32),
            pltpu.SemaphoreType.DMA,
            pltpu.SemaphoreType.DMA,
        ],
    )
    def att(asd_hbm, asdsw_hbm, den0_hbm, src_hbm, dst_hbm,
            ex_hbm, denp_hbm,
            sidx, didx, rows_s, rows_d, exbuf, dacc, sem1, sem2):
        c = lax.axis_index("c")
        s = lax.axis_index("s")
        wid = c * NS + s
        base = wid * PW

        @pl.when(s == 0)
        def _():
            pltpu.sync_copy(den0_hbm.at[c], dacc)

        plsc.subcore_barrier()

        def chunk_body(t, carry):
            off = base + t * CH
            pltpu.sync_copy(src_hbm.at[pl.ds(off, CH)], sidx.at[0])
            pltpu.sync_copy(dst_hbm.at[pl.ds(off, CH)], didx.at[0])
            cp1 = pltpu.async_copy(asd_hbm.at[sidx.at[0]], rows_s, sem1)
            cp2 = pltpu.async_copy(asdsw_hbm.at[didx.at[0]], rows_d, sem2)
            cp1.wait()
            cp2.wait()

            def edge_body(k, carry2):
                al = rows_s[k, :] + rows_d[k, :]
                exbuf[k, :] = jnp.exp(jnp.maximum(al, 0.2 * al))
                return carry2

            lax.fori_loop(0, CH, edge_body, 0, unroll=4)
            pltpu.sync_copy(exbuf, ex_hbm.at[pl.ds(off, CH)])
            pltpu.sync_copy(exbuf, dacc.at[didx.at[0]], add=True)
            return carry

        lax.fori_loop(0, chunks, chunk_body, 0)
        plsc.subcore_barrier()

        @pl.when(s == 0)
        def _():
            pltpu.sync_copy(dacc, denp_hbm.at[c])

    return att(asd, asdsw, deninit, src, dst)


def _sc_msg_call(xlflat, selfmsg, ex, src, dst, E, CH):
    PW = E // NS
    chunks = PW // CH
    mesh = plsc.VectorSubcoreMesh(core_axis_name="c", subcore_axis_name="s")

    @functools.partial(
        pl.kernel,
        out_type=jax.ShapeDtypeStruct((NC, N, 128), jnp.float32),
        mesh=mesh,
        compiler_params=pltpu.CompilerParams(use_tc_tiling_on_sc=False, needs_layout_passes=False),
        scratch_types=[
            pltpu.VMEM((1, CH), jnp.int32),
            pltpu.VMEM((1, CH), jnp.int32),
            pltpu.VMEM((CH, 16), jnp.float32),
            pltpu.VMEM((CH, 128), jnp.float32),
            pltpu.VMEM_SHARED((N, 128), jnp.float32),
            pltpu.SemaphoreType.DMA,
        ],
    )
    def msg(xl_hbm, sm_hbm, ex_hbm, src_hbm, dst_hbm, msg_hbm,
            sidx, didx, exb, grows, macc, sem1):
        c = lax.axis_index("c")
        s = lax.axis_index("s")
        base = s * PW

        @pl.when(s == 0)
        def _():
            pltpu.sync_copy(sm_hbm.at[c], macc)

        plsc.subcore_barrier()

        def chunk_body(t, carry):
            off = base + t * CH
            pltpu.sync_copy(src_hbm.at[pl.ds(off, CH)], sidx.at[0])
            pltpu.sync_copy(dst_hbm.at[pl.ds(off, CH)], didx.at[0])
            for j in range(CH // L):
                sidx[0, pl.ds(j * L, L)] = sidx[0, pl.ds(j * L, L)] + c * N
            pltpu.async_copy(xl_hbm.at[sidx.at[0]], grows, sem1).wait()
            pltpu.sync_copy(ex_hbm.at[pl.ds(off, CH)], exb)

            def edge_body(k, carry2):
                kf = jnp.full((L,), k, jnp.int32)
                for jh in range(4):
                    lane = jnp.full((L,), jh, jnp.int32) + 4 * c
                    w = plsc.load_gather(exb, [kf, lane])
                    grows[k, pl.ds(jh * 32, L)] = grows[k, pl.ds(jh * 32, L)] * w
                    grows[k, pl.ds(jh * 32 + L, L)] = (
                        grows[k, pl.ds(jh * 32 + L, L)] * w)
                return carry2

            lax.fori_loop(0, CH, edge_body, 0, unroll=2)
            pltpu.sync_copy(grows, macc.at[didx.at[0]], add=True)
            return carry

        lax.fori_loop(0, chunks, chunk_body, 0)
        plsc.subcore_barrier()

        @pl.when(s == 0)
        def _():
            pltpu.sync_copy(macc, msg_hbm.at[c])

    return msg(xlflat, selfmsg, ex, src, dst)


def _gat_post_body(h_ref, msg_ref, den_ref, gb_ref, n1g_ref, n1b_ref,
                   w1_ref, b1_ref, w2_ref, b2_ref, n2g_ref, n2b_ref, o_ref):
    num = jnp.concatenate([msg_ref[0], msg_ref[1]], axis=1)
    den = den_ref[0, :, :H] + den_ref[1, :, :H]
    r = 1.0 / (den + 1e-16)
    parts = []
    for hh in range(H):
        parts.append(num[:, hh * C:(hh + 1) * C] * r[:, hh:hh + 1])
    gat = jnp.concatenate(parts, axis=1) + gb_ref[...]
    h1 = _ln(h_ref[...] + gat, n1g_ref[...], n1b_ref[...])
    hf = jnp.maximum(h1 @ w1_ref[...] + b1_ref[...], 0.0) @ w2_ref[...] + b2_ref[...]
    o_ref[...] = _ln(h1 + hf, n2g_ref[...], n2b_ref[...])


def _gat_post_call(h, msgp, denp, p):
    return pl.pallas_call(
        _gat_post_body,
        grid=(N // BLK,),
        in_specs=[
            pl.BlockSpec((BLK, D), lambda i: (i, 0)),
            pl.BlockSpec((2, BLK, 128), lambda i: (0, i, 0)),
            pl.BlockSpec((2, BLK, 16), lambda i: (0, i, 0)),
            pl.BlockSpec((D,), lambda i: (0,)),
            pl.BlockSpec((D,), lambda i: (0,)),
            pl.BlockSpec((D,), lambda i: (0,)),
            pl.BlockSpec((D, FF), lambda i: (0, 0)),
            pl.BlockSpec((FF,), lambda i: (0,)),
            pl.BlockSpec((FF, D), lambda i: (0, 0)),
            pl.BlockSpec((D,), lambda i: (0,)),
            pl.BlockSpec((D,), lambda i: (0,)),
            pl.BlockSpec((D,), lambda i: (0,)),
        ],
        out_specs=pl.BlockSpec((BLK, D), lambda i: (i, 0)),
        out_shape=jax.ShapeDtypeStruct((N, D), jnp.float32),
    )(h, msgp, denp, p["gat_b"], p["n1_g"], p["n1_b"],
      p["ff_w1"], p["ff_b1"], p["ff_w2"], p["ff_b2"], p["n2_g"], p["n2_b"])


def _pred_pre_body(h_ref, w1_ref, b1_ref, uv_ref):
    hblk = h_ref[...]
    u = hblk @ w1_ref[0:D, :] + b1_ref[...]
    v = hblk @ w1_ref[D:2 * D, :]
    uv_ref[...] = jnp.concatenate([u[None], v[None]], 0)


def _pred_pre_call(h, w1, b1):
    return pl.pallas_call(
        _pred_pre_body,
        grid=(N // BLK,),
        in_specs=[
            pl.BlockSpec((BLK, D), lambda i: (i, 0)),
            pl.BlockSpec((2 * D, PRED), lambda i: (0, 0)),
            pl.BlockSpec((PRED,), lambda i: (0,)),
        ],
        out_specs=pl.BlockSpec((2, BLK, PRED), lambda i: (0, i, 0)),
        out_shape=jax.ShapeDtypeStruct((2, N, PRED), jnp.float32),
    )(h, w1, b1)


def _sc_pred_call(uvflat, src, dst, w2b, E, CH):
    PW = E // (NC * NS)
    chunks = PW // CH
    mesh = plsc.VectorSubcoreMesh(core_axis_name="c", subcore_axis_name="s")

    @functools.partial(
        pl.kernel,
        out_type=jax.ShapeDtypeStruct((E,), jnp.float32),
        mesh=mesh,
        compiler_params=pltpu.CompilerParams(use_tc_tiling_on_sc=False, needs_layout_passes=False),
        scratch_types=[
            pltpu.VMEM((1, CH), jnp.int32),
            pltpu.VMEM((1, CH), jnp.int32),
            pltpu.VMEM((CH, PRED), jnp.float32),
            pltpu.VMEM((CH, PRED), jnp.float32),
            pltpu.VMEM((1, CH), jnp.float32),
            pltpu.VMEM((272,), jnp.float32),
            pltpu.SemaphoreType.DMA,
            pltpu.SemaphoreType.DMA,
        ],
    )
    def pred(uv_hbm, src_hbm, dst_hbm, w2b_hbm, flow_hbm,
             sidx, didx, urows, vrows, fbuf, w2v, sem1, sem2):
        c = lax.axis_index("c")
        s = lax.axis_index("s")
        wid = c * NS + s
        base = wid * PW
        pltpu.sync_copy(w2b_hbm, w2v)
        b2s = w2v[pl.ds(PRED, L)][0]
        lanes = lax.iota(jnp.int32, L)

        def chunk_body(t, carry):
            off = base + t * CH
            pltpu.sync_copy(src_hbm.at[pl.ds(off, CH)], sidx.at[0])
            pltpu.sync_copy(dst_hbm.at[pl.ds(off, CH)], didx.at[0])
            for j in range(CH // L):
                didx[0, pl.ds(j * L, L)] = didx[0, pl.ds(j * L, L)] + N
            cp1 = pltpu.async_copy(uv_hbm.at[sidx.at[0]], urows, sem1)
            cp2 = pltpu.async_copy(uv_hbm.at[didx.at[0]], vrows, sem2)
            cp1.wait()
            cp2.wait()

            def group_body(g, carry2):
                res = jnp.zeros((L,), jnp.float32)
                for k16 in range(L):
                    k = g * L + k16
                    acc = jnp.zeros((L,), jnp.float32)
                    for j in range(PRED // L):
                        z = jnp.maximum(
                            urows[k, pl.ds(j * L, L)]
                            + vrows[k, pl.ds(j * L, L)], 0.0)
                        acc = acc + z * w2v[pl.ds(j * L, L)]
                    tot = jnp.maximum(jnp.sum(acc) + b2s, 0.0)
                    res = jnp.where(lanes == k16, tot, res)
                fbuf[0, pl.ds(g * L, L)] = res
                return carry2

            lax.fori_loop(0, CH // L, group_body, 0)
            pltpu.sync_copy(fbuf.at[0], flow_hbm.at[pl.ds(off, CH)])
            return carry

        lax.fori_loop(0, chunks, chunk_body, 0)

    return pred(uvflat, src, dst, w2b)


def _layer(h, src, dst, E, p):
    CH_att = 80 if (E // (NC * NS)) % 80 == 0 else 40
    CH_msg = 80 if (E // NS) % 80 == 0 else 40
    xl2, asd, asdsw, deninit, selfmsg = _gat_pre_call(
        h, p["gat_w"], p["att_src"].reshape(-1), p["att_dst"].reshape(-1))
    ex, denp = _sc_att_call(asd, asdsw, deninit, src, dst, E, CH_att)
    xlflat = xl2.reshape(2 * N, 128)
    msgp = _sc_msg_call(xlflat, selfmsg, ex, src, dst, E, CH_msg)
    return _gat_post_call(h, msgp, denp, p)


def kernel(x, edge_index, virtual_edge_index, params):
    src, dst = edge_index[0], edge_index[1]
    vsrc, vdst = virtual_edge_index[0], virtual_edge_index[1]
    h = _pre_call(x, params["pre_w"], params["pre_b"],
                  params["pre_g"], params["pre_bt"])
    for p in params["v"]:
        h = _layer(h, vsrc, vdst, E_V, p)
    for p in params["r"]:
        h = _layer(h, src, dst, E_R, p)
    pp = params["pred"]
    uv = _pred_pre_call(h, pp["w1"], pp["b1"])
    uvflat = uv.reshape(2 * N, PRED)
    w2b = jnp.concatenate(
        [pp["w2"][:, 0], pp["b2"], jnp.zeros((15,), jnp.float32)])
    flow = _sc_pred_call(uvflat, src, dst, w2b, E_R, 80)
    return flow


# speedup vs baseline: 27.7693x; 1.3240x over previous
import functools

import jax
import jax.numpy as jnp
from jax import lax
from jax.experimental import pallas as pl
from jax.experimental.pallas import tpu as pltpu
from jax.experimental.pallas import tpu_sc as plsc

N = 10000
D_IN = 128
D = 256
H = 8
C = 32
FF = 512
PRED = 256
E_R = 320000
E_V = 160000

NC = 2
NS = 16
L = 16

BLK = 1000


_PH = jax.lax.Precision.HIGHEST


def _ln(z, g, b, eps=1e-5):
    m = jnp.mean(z, axis=-1, keepdims=True)
    v = jnp.mean((z - m) ** 2, axis=-1, keepdims=True)
    return (z - m) / jnp.sqrt(v + eps) * g + b


def _pre_body(x_ref, w_ref, b_ref, g_ref, bt_ref, o_ref):
    z = jnp.maximum(jnp.dot(x_ref[...], w_ref[...], precision=_PH) + b_ref[...], 0.0)
    o_ref[...] = _ln(z, g_ref[...], bt_ref[...])


def _pre_call(x, pw, pb, pg, pbt):
    return pl.pallas_call(
        _pre_body,
        grid=(N // BLK,),
        in_specs=[
            pl.BlockSpec((BLK, D_IN), lambda i: (i, 0)),
            pl.BlockSpec((D_IN, D), lambda i: (0, 0)),
            pl.BlockSpec((D,), lambda i: (0,)),
            pl.BlockSpec((D,), lambda i: (0,)),
            pl.BlockSpec((D,), lambda i: (0,)),
        ],
        out_specs=pl.BlockSpec((BLK, D), lambda i: (i, 0)),
        out_shape=jax.ShapeDtypeStruct((N, D), jnp.float32),
    )(x, pw, pb, pg, pbt)


def _gat_pre_body(h_ref, w_ref, asrc_ref, adst_ref,
                  xl2_ref, asd_ref, asdsw_ref, den0_ref, sm_ref):
    xl = jnp.dot(h_ref[...], w_ref[...], precision=_PH)
    asrc = asrc_ref[...]
    adst = adst_ref[...]
    a_s_parts = []
    a_d_parts = []
    sm_parts = []
    for hh in range(H):
        xs = xl[:, hh * C:(hh + 1) * C]
        a_s_parts.append(jnp.sum(xs * asrc[hh * C:(hh + 1) * C][None, :],
                                 axis=1, keepdims=True))
        a_d_parts.append(jnp.sum(xs * adst[hh * C:(hh + 1) * C][None, :],
                                 axis=1, keepdims=True))
    a_s = jnp.concatenate(a_s_parts, axis=1)
    a_d = jnp.concatenate(a_d_parts, axis=1)
    al = a_s + a_d
    exl = jnp.exp(jnp.maximum(al, 0.2 * al))
    for hh in range(H):
        sm_parts.append(exl[:, hh:hh + 1] * xl[:, hh * C:(hh + 1) * C])
    sm = jnp.concatenate(sm_parts, axis=1)
    zeros8 = jnp.zeros_like(exl)
    exl16 = jnp.concatenate([exl, zeros8], axis=1)
    asd_ref[...] = jnp.concatenate([a_s, a_d], axis=1)
    asdsw_ref[...] = jnp.concatenate([a_d, a_s], axis=1)
    xl2_ref[...] = jnp.concatenate([xl[None, :, :128], xl[None, :, 128:]], 0)
    den0_ref[...] = jnp.concatenate([exl16[None], jnp.zeros_like(exl16)[None]], 0)
    sm_ref[...] = jnp.concatenate([sm[None, :, :128], sm[None, :, 128:]], 0)


def _gat_pre_call(h, gw, asrc, adst):
    return pl.pallas_call(
        _gat_pre_body,
        grid=(N // BLK,),
        in_specs=[
            pl.BlockSpec((BLK, D), lambda i: (i, 0)),
            pl.BlockSpec((D, D), lambda i: (0, 0)),
            pl.BlockSpec((D,), lambda i: (0,)),
            pl.BlockSpec((D,), lambda i: (0,)),
        ],
        out_specs=[
            pl.BlockSpec((2, BLK, 128), lambda i: (0, i, 0)),
            pl.BlockSpec((BLK, 16), lambda i: (i, 0)),
            pl.BlockSpec((BLK, 16), lambda i: (i, 0)),
            pl.BlockSpec((2, BLK, 16), lambda i: (0, i, 0)),
            pl.BlockSpec((2, BLK, 128), lambda i: (0, i, 0)),
        ],
        out_shape=[
            jax.ShapeDtypeStruct((2, N, 128), jnp.float32),
            jax.ShapeDtypeStruct((N, 16), jnp.float32),
            jax.ShapeDtypeStruct((N, 16), jnp.float32),
            jax.ShapeDtypeStruct((2, N, 16), jnp.float32),
            jax.ShapeDtypeStruct((2, N, 128), jnp.float32),
        ],
    )(h, gw, asrc, adst)


def _sc_att_call(asd, asdsw, deninit, src, dst, E, CH):
    PW = E // (NC * NS)
    chunks = PW // CH
    mesh = plsc.VectorSubcoreMesh(core_axis_name="c", subcore_axis_name="s")

    @functools.partial(
        pl.kernel,
        out_type=[
            jax.ShapeDtypeStruct((E, 16), jnp.float32),
            jax.ShapeDtypeStruct((NC, N, 16), jnp.float32),
        ],
        mesh=mesh,
        compiler_params=pltpu.CompilerParams(use_tc_tiling_on_sc=False, needs_layout_passes=False),
        scratch_types=[
            pltpu.VMEM((2, CH), jnp.int32),
            pltpu.VMEM((2, CH), jnp.int32),
            pltpu.VMEM((2, CH, 16), jnp.float32),
            pltpu.VMEM((2, CH, 16), jnp.float32),
            pltpu.VMEM((2, CH, 16), jnp.float32),
            pltpu.VMEM_SHARED((N, 16), jnp.float32),
            pltpu.SemaphoreType.DMA((2,)),
            pltpu.SemaphoreType.DMA((2,)),
            pltpu.SemaphoreType.DMA((2,)),
            pltpu.SemaphoreType.DMA((2,)),
        ],
    )
    def att(asd_hbm, asdsw_hbm, den0_hbm, src_hbm, dst_hbm,
            ex_hbm, denp_hbm,
            sidx, didx, rows_s, rows_d, exbuf, dacc, gsem1, gsem2, wsem, ssem):
        c = lax.axis_index("c")
        s = lax.axis_index("s")
        wid = c * NS + s
        base = wid * PW

        @pl.when(s == 0)
        def _():
            pltpu.sync_copy(den0_hbm.at[c], dacc)

        plsc.subcore_barrier()

        def stage_in(t, b):
            off = base + t * CH
            pltpu.sync_copy(src_hbm.at[pl.ds(off, CH)], sidx.at[b])
            pltpu.sync_copy(dst_hbm.at[pl.ds(off, CH)], didx.at[b])
            pltpu.async_copy(asd_hbm.at[sidx.at[b]], rows_s.at[b], gsem1.at[b])
            pltpu.async_copy(asdsw_hbm.at[didx.at[b]], rows_d.at[b], gsem2.at[b])

        def wait_out(b):
            pltpu.make_async_copy(
                exbuf.at[b], ex_hbm.at[pl.ds(0, CH)], wsem.at[b]).wait()
            pltpu.make_async_copy(
                exbuf.at[b], dacc.at[didx.at[b]], ssem.at[b]).wait()

        def body(t, b):
            b2 = 1 - b

            @pl.when(t + 1 < chunks)
            def _():
                @pl.when(t >= 1)
                def _():
                    wait_out(b2)
                stage_in(t + 1, b2)

            pltpu.make_async_copy(
                asd_hbm.at[sidx.at[b]], rows_s.at[b], gsem1.at[b]).wait()
            pltpu.make_async_copy(
                asdsw_hbm.at[didx.at[b]], rows_d.at[b], gsem2.at[b]).wait()

            def edge_body(k, carry2):
                al = rows_s[b, k, :] + rows_d[b, k, :]
                exbuf[b, k, :] = jnp.exp(jnp.maximum(al, 0.2 * al))
                return carry2

            lax.fori_loop(0, CH, edge_body, 0, unroll=4)
            off = base + t * CH
            pltpu.async_copy(exbuf.at[b], ex_hbm.at[pl.ds(off, CH)], wsem.at[b])
            pltpu.async_copy(exbuf.at[b], dacc.at[didx.at[b]], ssem.at[b],
                             add=True)

        stage_in(0, 0)

        def outer(t2, carry):
            body(2 * t2, 0)
            body(2 * t2 + 1, 1)
            return carry

        lax.fori_loop(0, chunks // 2, outer, 0)
        if chunks % 2:
            body(chunks - 1, 0)
        wait_out(0)
        wait_out(1)
        plsc.subcore_barrier()

        @pl.when(s == 0)
        def _():
            pltpu.sync_copy(dacc, denp_hbm.at[c])

    return att(asd, asdsw, deninit, src, dst)


def _sc_msg_call(xlflat, selfmsg, ex, src, dst, E, CH):
    PW = E // NS
    chunks = PW // CH
    mesh = plsc.VectorSubcoreMesh(core_axis_name="c", subcore_axis_name="s")

    @functools.partial(
        pl.kernel,
        out_type=jax.ShapeDtypeStruct((NC, N, 128), jnp.float32),
        mesh=mesh,
        compiler_params=pltpu.CompilerParams(use_tc_tiling_on_sc=False, needs_layout_passes=False),
        scratch_types=[
            pltpu.VMEM((2, CH), jnp.int32),
            pltpu.VMEM((2, CH), jnp.int32),
            pltpu.VMEM((2, CH, 16), jnp.float32),
            pltpu.VMEM((2, CH, 128), jnp.float32),
            pltpu.VMEM_SHARED((N, 128), jnp.float32),
            pltpu.SemaphoreType.DMA((2,)),
            pltpu.SemaphoreType.DMA((2,)),
            pltpu.SemaphoreType.DMA((2,)),
        ],
    )
    def msg(xl_hbm, sm_hbm, ex_hbm, src_hbm, dst_hbm, msg_hbm,
            sidx, didx, exb, grows, macc, gsem, esem, ssem):
        c = lax.axis_index("c")
        s = lax.axis_index("s")
        base = s * PW

        @pl.when(s == 0)
        def _():
            pltpu.sync_copy(sm_hbm.at[c], macc)

        plsc.subcore_barrier()

        def stage_in(t, b):
            off = base + t * CH
            pltpu.sync_copy(src_hbm.at[pl.ds(off, CH)], sidx.at[b])
            pltpu.sync_copy(dst_hbm.at[pl.ds(off, CH)], didx.at[b])
            for j in range(CH // L):
                sidx[b, pl.ds(j * L, L)] = sidx[b, pl.ds(j * L, L)] + c * N
            pltpu.async_copy(xl_hbm.at[sidx.at[b]], grows.at[b], gsem.at[b])
            pltpu.async_copy(ex_hbm.at[pl.ds(off, CH)], exb.at[b], esem.at[b])

        def wait_out(b):
            pltpu.make_async_copy(
                grows.at[b], macc.at[didx.at[b]], ssem.at[b]).wait()

        def body(t, b):
            b2 = 1 - b

            @pl.when(t + 1 < chunks)
            def _():
                @pl.when(t >= 1)
                def _():
                    wait_out(b2)
                stage_in(t + 1, b2)

            pltpu.make_async_copy(
                xl_hbm.at[sidx.at[b]], grows.at[b], gsem.at[b]).wait()
            pltpu.make_async_copy(
                ex_hbm.at[pl.ds(0, CH)], exb.at[b], esem.at[b]).wait()

            def edge_body(k, carry2):
                kf = jnp.full((L,), k, jnp.int32)
                for jh in range(4):
                    lane = jnp.full((L,), jh, jnp.int32) + 4 * c
                    w = plsc.load_gather(exb.at[b], [kf, lane])
                    grows[b, k, pl.ds(jh * 32, L)] = (
                        grows[b, k, pl.ds(jh * 32, L)] * w)
                    grows[b, k, pl.ds(jh * 32 + L, L)] = (
                        grows[b, k, pl.ds(jh * 32 + L, L)] * w)
                return carry2

            lax.fori_loop(0, CH, edge_body, 0, unroll=2)
            pltpu.async_copy(grows.at[b], macc.at[didx.at[b]], ssem.at[b],
                             add=True)

        stage_in(0, 0)

        def outer(t2, carry):
            body(2 * t2, 0)
            body(2 * t2 + 1, 1)
            return carry

        lax.fori_loop(0, chunks // 2, outer, 0)
        if chunks % 2:
            body(chunks - 1, 0)
        wait_out(0)
        wait_out(1)
        plsc.subcore_barrier()

        @pl.when(s == 0)
        def _():
            pltpu.sync_copy(macc, msg_hbm.at[c])

    return msg(xlflat, selfmsg, ex, src, dst)


def _gat_post_body(h_ref, msg_ref, den_ref, gb_ref, n1g_ref, n1b_ref,
                   w1_ref, b1_ref, w2_ref, b2_ref, n2g_ref, n2b_ref, o_ref):
    num = jnp.concatenate([msg_ref[0], msg_ref[1]], axis=1)
    den = den_ref[0, :, :H] + den_ref[1, :, :H]
    r = 1.0 / (den + 1e-16)
    parts = []
    for hh in range(H):
        parts.append(num[:, hh * C:(hh + 1) * C] * r[:, hh:hh + 1])
    gat = jnp.concatenate(parts, axis=1) + gb_ref[...]
    h1 = _ln(h_ref[...] + gat, n1g_ref[...], n1b_ref[...])
    hf = jnp.dot(jnp.maximum(jnp.dot(h1, w1_ref[...], precision=_PH) + b1_ref[...], 0.0), w2_ref[...], precision=_PH) + b2_ref[...]
    o_ref[...] = _ln(h1 + hf, n2g_ref[...], n2b_ref[...])


def _gat_post_call(h, msgp, denp, p):
    return pl.pallas_call(
        _gat_post_body,
        grid=(N // BLK,),
        in_specs=[
            pl.BlockSpec((BLK, D), lambda i: (i, 0)),
            pl.BlockSpec((2, BLK, 128), lambda i: (0, i, 0)),
            pl.BlockSpec((2, BLK, 16), lambda i: (0, i, 0)),
            pl.BlockSpec((D,), lambda i: (0,)),
            pl.BlockSpec((D,), lambda i: (0,)),
            pl.BlockSpec((D,), lambda i: (0,)),
            pl.BlockSpec((D, FF), lambda i: (0, 0)),
            pl.BlockSpec((FF,), lambda i: (0,)),
            pl.BlockSpec((FF, D), lambda i: (0, 0)),
            pl.BlockSpec((D,), lambda i: (0,)),
            pl.BlockSpec((D,), lambda i: (0,)),
            pl.BlockSpec((D,), lambda i: (0,)),
        ],
        out_specs=pl.BlockSpec((BLK, D), lambda i: (i, 0)),
        out_shape=jax.ShapeDtypeStruct((N, D), jnp.float32),
    )(h, msgp, denp, p["gat_b"], p["n1_g"], p["n1_b"],
      p["ff_w1"], p["ff_b1"], p["ff_w2"], p["ff_b2"], p["n2_g"], p["n2_b"])


def _pred_pre_body(h_ref, w1_ref, b1_ref, uv_ref):
    hblk = h_ref[...]
    u = jnp.dot(hblk, w1_ref[0:D, :], precision=_PH) + b1_ref[...]
    v = jnp.dot(hblk, w1_ref[D:2 * D, :], precision=_PH)
    uv_ref[...] = jnp.concatenate([u[None], v[None]], 0)


def _pred_pre_call(h, w1, b1):
    return pl.pallas_call(
        _pred_pre_body,
        grid=(N // BLK,),
        in_specs=[
            pl.BlockSpec((BLK, D), lambda i: (i, 0)),
            pl.BlockSpec((2 * D, PRED), lambda i: (0, 0)),
            pl.BlockSpec((PRED,), lambda i: (0,)),
        ],
        out_specs=pl.BlockSpec((2, BLK, PRED), lambda i: (0, i, 0)),
        out_shape=jax.ShapeDtypeStruct((2, N, PRED), jnp.float32),
    )(h, w1, b1)


def _sc_pred_call(uvflat, src, dst, w2b, E, CH):
    PW = E // (NC * NS)
    chunks = PW // CH
    mesh = plsc.VectorSubcoreMesh(core_axis_name="c", subcore_axis_name="s")

    @functools.partial(
        pl.kernel,
        out_type=jax.ShapeDtypeStruct((E,), jnp.float32),
        mesh=mesh,
        compiler_params=pltpu.CompilerParams(use_tc_tiling_on_sc=False, needs_layout_passes=False),
        scratch_types=[
            pltpu.VMEM((2, CH), jnp.int32),
            pltpu.VMEM((2, CH), jnp.int32),
            pltpu.VMEM((2, CH, PRED), jnp.float32),
            pltpu.VMEM((2, CH, PRED), jnp.float32),
            pltpu.VMEM((2, CH), jnp.float32),
            pltpu.VMEM((272,), jnp.float32),
            pltpu.SemaphoreType.DMA((2,)),
            pltpu.SemaphoreType.DMA((2,)),
            pltpu.SemaphoreType.DMA((2,)),
        ],
    )
    def pred(uv_hbm, src_hbm, dst_hbm, w2b_hbm, flow_hbm,
             sidx, didx, urows, vrows, fbuf, w2v, gsem1, gsem2, wsem):
        c = lax.axis_index("c")
        s = lax.axis_index("s")
        wid = c * NS + s
        base = wid * PW
        pltpu.sync_copy(w2b_hbm, w2v)
        b2s = w2v[pl.ds(PRED, L)][0]
        lanes = lax.iota(jnp.int32, L)

        def stage_in(t, b):
            off = base + t * CH
            pltpu.sync_copy(src_hbm.at[pl.ds(off, CH)], sidx.at[b])
            pltpu.sync_copy(dst_hbm.at[pl.ds(off, CH)], didx.at[b])
            for j in range(CH // L):
                didx[b, pl.ds(j * L, L)] = didx[b, pl.ds(j * L, L)] + N
            pltpu.async_copy(uv_hbm.at[sidx.at[b]], urows.at[b], gsem1.at[b])
            pltpu.async_copy(uv_hbm.at[didx.at[b]], vrows.at[b], gsem2.at[b])

        def wait_out(b):
            pltpu.make_async_copy(
                fbuf.at[b], flow_hbm.at[pl.ds(0, CH)], wsem.at[b]).wait()

        def body(t, b):
            b2 = 1 - b

            @pl.when(t + 1 < chunks)
            def _():
                @pl.when(t >= 1)
                def _():
                    wait_out(b2)
                stage_in(t + 1, b2)

            pltpu.make_async_copy(
                uv_hbm.at[sidx.at[b]], urows.at[b], gsem1.at[b]).wait()
            pltpu.make_async_copy(
                uv_hbm.at[didx.at[b]], vrows.at[b], gsem2.at[b]).wait()

            def group_body(g, carry2):
                res = jnp.zeros((L,), jnp.float32)
                for k16 in range(L):
                    k = g * L + k16
                    acc = jnp.zeros((L,), jnp.float32)
                    for j in range(PRED // L):
                        z = jnp.maximum(
                            urows[b, k, pl.ds(j * L, L)]
                            + vrows[b, k, pl.ds(j * L, L)], 0.0)
                        acc = acc + z * w2v[pl.ds(j * L, L)]
                    tot = jnp.maximum(jnp.sum(acc) + b2s, 0.0)
                    res = jnp.where(lanes == k16, tot, res)
                fbuf[b, pl.ds(g * L, L)] = res
                return carry2

            lax.fori_loop(0, CH // L, group_body, 0)
            off = base + t * CH
            pltpu.async_copy(fbuf.at[b], flow_hbm.at[pl.ds(off, CH)],
                             wsem.at[b])

        stage_in(0, 0)

        def outer(t2, carry):
            body(2 * t2, 0)
            body(2 * t2 + 1, 1)
            return carry

        lax.fori_loop(0, chunks // 2, outer, 0)
        if chunks % 2:
            body(chunks - 1, 0)
        wait_out(0)
        wait_out(1)

    return pred(uvflat, src, dst, w2b)


def _layer(h, src, dst, E, p):
    CH_att = 80 if (E // (NC * NS)) % 80 == 0 else 40
    CH_msg = 80 if (E // NS) % 80 == 0 else 40
    xl2, asd, asdsw, deninit, selfmsg = _gat_pre_call(
        h, p["gat_w"], p["att_src"].reshape(-1), p["att_dst"].reshape(-1))
    ex, denp = _sc_att_call(asd, asdsw, deninit, src, dst, E, CH_att)
    xlflat = xl2.reshape(2 * N, 128)
    msgp = _sc_msg_call(xlflat, selfmsg, ex, src, dst, E, CH_msg)
    return _gat_post_call(h, msgp, denp, p)


def kernel(x, edge_index, virtual_edge_index, params):
    src, dst = edge_index[0], edge_index[1]
    vsrc, vdst = virtual_edge_index[0], virtual_edge_index[1]
    h = _pre_call(x, params["pre_w"], params["pre_b"],
                  params["pre_g"], params["pre_bt"])
    for p in params["v"]:
        h = _layer(h, vsrc, vdst, E_V, p)
    for p in params["r"]:
        h = _layer(h, src, dst, E_R, p)
    pp = params["pred"]
    uv = _pred_pre_call(h, pp["w1"], pp["b1"])
    uvflat = uv.reshape(2 * N, PRED)
    w2b = jnp.concatenate(
        [pp["w2"][:, 0], pp["b2"], jnp.zeros((15,), jnp.float32)])
    flow = _sc_pred_call(uvflat, src, dst, w2b, E_R, 80)
    return flow
